# baseline (speedup 1.0000x reference)
import functools

import jax
import jax.numpy as jnp
from jax import lax
from jax.experimental import pallas as pl
from jax.experimental.pallas import tpu as pltpu
from jax.experimental.pallas import tpu_sc as plsc

H = 128
NC = 2
NS = 16
NW = NC * NS
C = 80


def _emb_body(nchunk, seq, token_hbm, pos_hbm, delta_hbm, segf_hbm, tidx_hbm,
              out_hbm, tix_all, seg_all, pos_v, delta_v, bufs, sems):
    a0, o0, a1, o1 = bufs
    sg0, sg1, st0, st1 = sems
    wid = lax.axis_index("s") * NC + lax.axis_index("c")
    base = wid * (nchunk * C)

    pltpu.sync_copy(pos_hbm, pos_v)
    pltpu.sync_copy(delta_hbm, delta_v)
    pltpu.sync_copy(tidx_hbm.at[wid], tix_all)
    pltpu.sync_copy(segf_hbm.at[wid], seg_all)

    def start_gather(g, buf_a, sem):
        pltpu.async_copy(token_hbm.at[tix_all.at[g]], buf_a, sem)

    def wait_gather(g, buf_a, sem):
        pltpu.make_async_copy(token_hbm.at[tix_all.at[g]], buf_a, sem).wait()

    def out_slice(g):
        return out_hbm.at[pl.ds(base + g * C, C)]

    _gdims = lax.GatherDimensionNumbers(
        offset_dims=(), collapsed_slice_dims=(0,), start_index_map=(0,))

    def _vreg_gather(vec, idx):
        return lax.gather(vec, idx[:, None], _gdims, (1,),
                          mode=lax.GatherScatterMode.PROMISE_IN_BOUNDS)

    def add_chunk(g, buf_a, buf_o):
        s_off = lax.rem(base + g * C, seq)
        dv = [delta_v[pl.ds(j * 16, 16)] for j in range(H // 16)]

        @plsc.parallel_loop(0, C // 16, step=1, unroll=1)
        def _(rb):
            sv = seg_all[g, pl.ds(rb * 16, 16)]
            for l in range(16):
                lane = lax.broadcast_in_dim(jnp.int32(l), (16,), ())
                splat = _vreg_gather(sv, lane)
                r = rb * 16 + l
                pr = s_off + r
                for j in range(H // 16):
                    sl = pl.ds(j * 16, 16)
                    buf_o[r, sl] = (buf_a[r, sl] + pos_v[pr, sl]
                                    + splat * dv[j])

    start_gather(0, a0, sg0)
    start_gather(1, a1, sg1)

    def pair(k, carry):
        g0 = 2 * k
        g1 = g0 + 1

        wait_gather(g0, a0, sg0)

        @pl.when(k > 0)
        def _():
            pltpu.make_async_copy(o0, out_slice(g0 - 2), st0).wait()

        add_chunk(g0, a0, o0)

        @pl.when(k < nchunk // 2 - 1)
        def _():
            start_gather(g0 + 2, a0, sg0)

        pltpu.async_copy(o0, out_slice(g0), st0)

        wait_gather(g1, a1, sg1)

        @pl.when(k > 0)
        def _():
            pltpu.make_async_copy(o1, out_slice(g1 - 2), st1).wait()

        add_chunk(g1, a1, o1)

        @pl.when(k < nchunk // 2 - 1)
        def _():
            start_gather(g1 + 2, a1, sg1)

        pltpu.async_copy(o1, out_slice(g1), st1)
        return carry

    lax.fori_loop(0, nchunk // 2, pair, 0, unroll=False)

    pltpu.make_async_copy(o0, out_slice(nchunk - 2), st0).wait()
    pltpu.make_async_copy(o1, out_slice(nchunk - 1), st1).wait()


def kernel(sentences, segments, token_table, segment_table, positional_embedding):
    batch, seq = sentences.shape
    bs = batch * seq
    assert bs % (NW * C) == 0
    nchunk = bs // (NW * C)
    assert nchunk % 2 == 0

    pos_used = positional_embedding[0, :seq, :]
    pos_ext = (jnp.concatenate([pos_used, pos_used[:C]], axis=0)
               + segment_table[0][None, :])
    delta = segment_table[1] - segment_table[0]
    segf = segments.reshape(NW, nchunk, C).astype(jnp.float32)
    tidx = sentences.reshape(NW, nchunk, C).astype(jnp.int32)

    mesh = plsc.VectorSubcoreMesh(core_axis_name="c", subcore_axis_name="s")
    run = pl.kernel(
        functools.partial(_emb_body, nchunk, seq),
        out_type=jax.ShapeDtypeStruct((bs, H), jnp.float32),
        mesh=mesh,
        scratch_types=[
            pltpu.VMEM((nchunk, C), jnp.int32),
            pltpu.VMEM((nchunk, C), jnp.float32),
            pltpu.VMEM((seq + C, H), jnp.float32),
            pltpu.VMEM((H,), jnp.float32),
            tuple(pltpu.VMEM((C, H), jnp.float32) for _ in range(4)),
            tuple(pltpu.SemaphoreType.DMA for _ in range(4)),
        ],
    )
    out = run(token_table, pos_ext, delta, segf, tidx)
    return out.reshape(batch, seq, H)


# speedup vs baseline: 1.5217x; 1.5217x over previous
import functools

import jax
import jax.numpy as jnp
from jax import lax
from jax.experimental import pallas as pl
from jax.experimental.pallas import tpu as pltpu
from jax.experimental.pallas import tpu_sc as plsc

H = 128
NC = 2
NS = 16
NW = NC * NS
C = 80


def _emb_body(nchunk, seq, token_hbm, pos_hbm, delta_hbm, segb_hbm, tidx_hbm,
              out_hbm, tix_all, pos_v, delta_v, bufs, sems):
    a0, o0, sb0, a1, o1, sb1 = bufs
    sg0, sg1, st0, st1, sb_sem0, sb_sem1 = sems
    wid = lax.axis_index("s") * NC + lax.axis_index("c")
    base = wid * (nchunk * C)

    pltpu.sync_copy(pos_hbm, pos_v)
    pltpu.sync_copy(delta_hbm, delta_v)
    pltpu.sync_copy(tidx_hbm.at[wid], tix_all)

    def start_gather(g, buf_a, sb, sem, sbsem):
        pltpu.async_copy(token_hbm.at[tix_all.at[g]], buf_a, sem)
        pltpu.async_copy(segb_hbm.at[wid].at[g], sb, sbsem)

    def wait_segb(g, sb, sbsem):
        pltpu.make_async_copy(segb_hbm.at[wid].at[g], sb, sbsem).wait()

    def wait_gather(g, buf_a, sem):
        pltpu.make_async_copy(token_hbm.at[tix_all.at[g]], buf_a, sem).wait()

    def out_slice(g):
        return out_hbm.at[pl.ds(base + g * C, C)]

    def phase_a(g, sb, buf_o):
        s_off = lax.rem(base + g * C, seq)
        dv = [delta_v[pl.ds(j * 16, 16)] for j in range(H // 16)]

        @plsc.parallel_loop(0, C, step=1, unroll=4)
        def _(r):
            seg_splat = sb[r, pl.ds(0, 16)]
            pr = s_off + r
            for j in range(H // 16):
                sl = pl.ds(j * 16, 16)
                buf_o[r, sl] = pos_v[pr, sl] + seg_splat * dv[j]

    def phase_b(buf_a, buf_o):
        @plsc.parallel_loop(0, C, step=1, unroll=4)
        def _(r):
            for j in range(H // 16):
                sl = pl.ds(j * 16, 16)
                plsc.addupdate(buf_o.at[r, sl], buf_a[r, sl])

    start_gather(0, a0, sb0, sg0, sb_sem0)
    start_gather(1, a1, sb1, sg1, sb_sem1)

    def pair(k, carry):
        g0 = 2 * k
        g1 = g0 + 1

        @pl.when(k > 0)
        def _():
            pltpu.make_async_copy(o0, out_slice(g0 - 2), st0).wait()

        wait_segb(g0, sb0, sb_sem0)
        phase_a(g0, sb0, o0)
        wait_gather(g0, a0, sg0)
        phase_b(a0, o0)

        @pl.when(k < nchunk // 2 - 1)
        def _():
            start_gather(g0 + 2, a0, sb0, sg0, sb_sem0)

        pltpu.async_copy(o0, out_slice(g0), st0)

        @pl.when(k > 0)
        def _():
            pltpu.make_async_copy(o1, out_slice(g1 - 2), st1).wait()

        wait_segb(g1, sb1, sb_sem1)
        phase_a(g1, sb1, o1)
        wait_gather(g1, a1, sg1)
        phase_b(a1, o1)

        @pl.when(k < nchunk // 2 - 1)
        def _():
            start_gather(g1 + 2, a1, sb1, sg1, sb_sem1)

        pltpu.async_copy(o1, out_slice(g1), st1)
        return carry

    lax.fori_loop(0, nchunk // 2, pair, 0, unroll=False)

    pltpu.make_async_copy(o0, out_slice(nchunk - 2), st0).wait()
    pltpu.make_async_copy(o1, out_slice(nchunk - 1), st1).wait()


def kernel(sentences, segments, token_table, segment_table, positional_embedding):
    batch, seq = sentences.shape
    bs = batch * seq
    assert bs % (NW * C) == 0
    nchunk = bs // (NW * C)
    assert nchunk % 2 == 0

    pos_used = positional_embedding[0, :seq, :]
    pos_ext = (jnp.concatenate([pos_used, pos_used[:C]], axis=0)
               + segment_table[0][None, :])
    delta = segment_table[1] - segment_table[0]
    segb = jnp.broadcast_to(
        segments.reshape(NW, nchunk, C, 1).astype(jnp.float32),
        (NW, nchunk, C, 16))
    tidx = sentences.reshape(NW, nchunk, C).astype(jnp.int32)

    mesh = plsc.VectorSubcoreMesh(core_axis_name="c", subcore_axis_name="s")
    run = pl.kernel(
        functools.partial(_emb_body, nchunk, seq),
        out_type=jax.ShapeDtypeStruct((bs, H), jnp.float32),
        mesh=mesh,
        scratch_types=[
            pltpu.VMEM((nchunk, C), jnp.int32),
            pltpu.VMEM((seq + C, H), jnp.float32),
            pltpu.VMEM((H,), jnp.float32),
            tuple([pltpu.VMEM((C, H), jnp.float32),
                   pltpu.VMEM((C, H), jnp.float32),
                   pltpu.VMEM((C, 16), jnp.float32)] * 2),
            tuple(pltpu.SemaphoreType.DMA for _ in range(6)),
        ],
    )
    out = run(token_table, pos_ext, delta, segb, tidx)
    return out.reshape(batch, seq, H)


# speedup vs baseline: 2.0257x; 1.3313x over previous
import functools

import jax
import jax.numpy as jnp
from jax import lax
from jax.experimental import pallas as pl
from jax.experimental.pallas import tpu as pltpu
from jax.experimental.pallas import tpu_sc as plsc

H = 128
NC = 2
NS = 16
NW = NC * NS
C = 64
NSETS = 4


def _emb_body(nchunk, seq, token_hbm, pos_hbm, delta_hbm, segb_hbm, tidx_hbm,
              out_hbm, tix_all, pos_v, delta_v, bufs, gsems, bsems, ssems):
    wid = lax.axis_index("s") * NC + lax.axis_index("c")
    base = wid * (nchunk * C)

    pltpu.sync_copy(pos_hbm, pos_v)
    pltpu.sync_copy(delta_hbm, delta_v)
    pltpu.sync_copy(tidx_hbm.at[wid], tix_all)

    def start_gather(g, s):
        a, _, sb = bufs[s]
        pltpu.async_copy(token_hbm.at[tix_all.at[g]], a, gsems[s])
        pltpu.async_copy(segb_hbm.at[wid].at[g], sb, bsems[s])

    def out_slice(g):
        return out_hbm.at[pl.ds(base + g * C, C)]

    def add_chunk(g, s):
        a, o, sb = bufs[s]
        pltpu.make_async_copy(segb_hbm.at[wid].at[g], sb, bsems[s]).wait()
        pltpu.make_async_copy(token_hbm.at[tix_all.at[g]], a, gsems[s]).wait()
        s_off = lax.rem(base + g * C, seq)
        dv = [delta_v[pl.ds(j * 16, 16)] for j in range(H // 16)]

        @plsc.parallel_loop(0, C, step=1, unroll=4)
        def _(r):
            seg_splat = sb[pl.ds(r * 16, 16)]
            pr = s_off + r
            for j in range(H // 16):
                sl = pl.ds(j * 16, 16)
                o[r, sl] = a[r, sl] + pos_v[pr, sl] + seg_splat * dv[j]

    for s in range(NSETS):
        start_gather(s, s)

    def quad(q, carry):
        for s in range(NSETS):
            g = NSETS * q + s
            _, o, _ = bufs[s]

            @pl.when(q > 0)
            def _():
                pltpu.make_async_copy(o, out_slice(g - NSETS),
                                      ssems[s]).wait()

            add_chunk(g, s)

            @pl.when(q < nchunk // NSETS - 1)
            def _():
                start_gather(g + NSETS, s)

            pltpu.async_copy(o, out_slice(g), ssems[s])
        return carry

    lax.fori_loop(0, nchunk // NSETS, quad, 0, unroll=False)

    for s in range(NSETS):
        _, o, _ = bufs[s]
        pltpu.make_async_copy(o, out_slice(nchunk - NSETS + s), ssems[s]).wait()


def kernel(sentences, segments, token_table, segment_table, positional_embedding):
    batch, seq = sentences.shape
    bs = batch * seq
    assert bs % (NW * C) == 0
    nchunk = bs // (NW * C)
    assert nchunk % NSETS == 0

    pos_used = positional_embedding[0, :seq, :]
    pos_ext = (jnp.concatenate([pos_used, pos_used[:C]], axis=0)
               + segment_table[0][None, :])
    delta = segment_table[1] - segment_table[0]
    segb = jnp.broadcast_to(
        segments.reshape(NW, nchunk, C, 1).astype(jnp.float32),
        (NW, nchunk, C, 16)).reshape(NW, nchunk, C * 16)
    tidx = sentences.reshape(NW, nchunk, C).astype(jnp.int32)

    mesh = plsc.VectorSubcoreMesh(core_axis_name="c", subcore_axis_name="s")
    run = pl.kernel(
        functools.partial(_emb_body, nchunk, seq),
        out_type=jax.ShapeDtypeStruct((bs, H), jnp.float32),
        mesh=mesh,
        scratch_types=[
            pltpu.VMEM((nchunk, C), jnp.int32),
            pltpu.VMEM((seq + C, H), jnp.float32),
            pltpu.VMEM((H,), jnp.float32),
            tuple(tuple([pltpu.VMEM((C, H), jnp.float32),
                         pltpu.VMEM((C, H), jnp.float32),
                         pltpu.VMEM((C * 16,), jnp.float32)])
                  for _ in range(NSETS)),
            tuple(pltpu.SemaphoreType.DMA for _ in range(NSETS)),
            tuple(pltpu.SemaphoreType.DMA for _ in range(NSETS)),
            tuple(pltpu.SemaphoreType.DMA for _ in range(NSETS)),
        ],
    )
    out = run(token_table, pos_ext, delta, segb, tidx)
    return out.reshape(batch, seq, H)
